# baseline (speedup 1.0000x reference)
import jax
import jax.numpy as jnp
from jax.experimental import pallas as pl
from jax.experimental.pallas import tpu as pltpu

_T = 2048
_D = 1024
_E = 8
_NQ = 2
_QW = _D // _NQ


_NX = 4
_XR = _T // _NX


def _fused_kernel(x_ref, wa_ref, bias_ref, ws_ref, bs_ref, out_ref,
                  xb_ref, w_ref, acc_ref, xf_ref, sem, sem_x):
    e = pl.program_id(0)

    def _steps(init, start_copies, w=None, xb=None):
        if w is None:
            w = w_ref[...]
        if xb is None:
            xb = xb_ref[...]
        lane = jax.lax.broadcasted_iota(jnp.int32, w.shape, 1)
        we = jnp.sum(jnp.where(lane == e, w, 0.0), axis=1, keepdims=True)
        for q in range(_NQ):
            qsl = pl.ds(q * _QW, _QW)
            wq = ws_ref[0, :, qsl].astype(jnp.bfloat16)
            h = jnp.dot(xb, wq, preferred_element_type=jnp.float32)
            c = we * jnp.tanh(h + bs_ref[0, :, qsl])
            if init:
                acc_ref[:, qsl] = c
            else:
                acc_ref[:, qsl] = acc_ref[:, qsl] + c
            if start_copies:
                pltpu.make_async_copy(acc_ref.at[:, qsl], out_ref.at[:, qsl],
                                      sem.at[q]).start()
        if start_copies:
            for q in range(_NQ):
                qsl = pl.ds(q * _QW, _QW)
                pltpu.make_async_copy(acc_ref.at[:, qsl], out_ref.at[:, qsl],
                                      sem.at[q]).wait()

    @pl.when(e == 0)
    def _first():
        for m in range(_NX):
            msl = pl.ds(m * _XR, _XR)
            pltpu.make_async_copy(x_ref.at[msl, :], xf_ref.at[msl, :],
                                  sem_x.at[m]).start()
        for m in range(_NX):
            msl = pl.ds(m * _XR, _XR)
            pltpu.make_async_copy(x_ref.at[msl, :], xf_ref.at[msl, :],
                                  sem_x.at[m]).wait()
            x32 = xf_ref[msl, :]
            logits = jnp.dot(x32, wa_ref[...],
                             preferred_element_type=jnp.float32) + bias_ref[...]
            w_ref[msl, :] = jax.nn.softmax(logits, axis=-1)
            xb_ref[msl, :] = x32.astype(jnp.bfloat16)
        _steps(True, False)

    @pl.when(jnp.logical_and(e > 0, e < _E - 1))
    def _main():
        _steps(False, False)

    @pl.when(e == _E - 1)
    def _last():
        _steps(False, True)


def kernel(x, W_attn, b_attn, adaptive_bias, W_s, b_s):
    bias = (b_attn + adaptive_bias).reshape(1, _E)
    return pl.pallas_call(
        _fused_kernel,
        grid=(_E,),
        in_specs=[
            pl.BlockSpec(memory_space=pltpu.MemorySpace.HBM),
            pl.BlockSpec((_D, _E), lambda e: (0, 0)),
            pl.BlockSpec((1, _E), lambda e: (0, 0)),
            pl.BlockSpec((1, _D, _D), lambda e: (e, 0, 0)),
            pl.BlockSpec((1, 1, _D), lambda e: (e, 0, 0)),
        ],
        out_specs=pl.BlockSpec(memory_space=pltpu.MemorySpace.HBM),
        out_shape=jax.ShapeDtypeStruct((_T, _D), jnp.float32),
        scratch_shapes=[
            pltpu.VMEM((_T, _D), jnp.bfloat16),
            pltpu.VMEM((_T, _E), jnp.float32),
            pltpu.VMEM((_T, _D), jnp.float32),
            pltpu.VMEM((_T, _D), jnp.float32),
            pltpu.SemaphoreType.DMA((_NQ,)),
            pltpu.SemaphoreType.DMA((_NX,)),
        ],
        compiler_params=pltpu.CompilerParams(
            dimension_semantics=("arbitrary",),
        ),
    )(x, W_attn, bias, W_s, b_s.reshape(_E, 1, _D))


# speedup vs baseline: 1.0334x; 1.0334x over previous
import jax
import jax.numpy as jnp
from jax.experimental import pallas as pl
from jax.experimental.pallas import tpu as pltpu

_T = 2048
_D = 1024
_E = 8
_NQ = 2
_QW = _D // _NQ


def _fused_kernel(x_ref, wa_ref, bias_ref, ws_ref, bs_ref, out_ref,
                  xb_ref, w_ref, acc_ref, sem):
    e = pl.program_id(0)

    def _steps(init, start_copies, w=None, xb=None):
        if w is None:
            w = w_ref[...]
        if xb is None:
            xb = xb_ref[...]
        lane = jax.lax.broadcasted_iota(jnp.int32, w.shape, 1)
        we = jnp.sum(jnp.where(lane == e, w, 0.0), axis=1, keepdims=True)
        for q in range(_NQ):
            qsl = pl.ds(q * _QW, _QW)
            wq = ws_ref[0, :, qsl].astype(jnp.bfloat16)
            h = jnp.dot(xb, wq, preferred_element_type=jnp.float32)
            c = we * jnp.tanh(h + bs_ref[0, :, qsl])
            if init:
                acc_ref[:, qsl] = c
            else:
                acc_ref[:, qsl] = acc_ref[:, qsl] + c
            if start_copies:
                pltpu.make_async_copy(acc_ref.at[:, qsl], out_ref.at[:, qsl],
                                      sem.at[q]).start()
        if start_copies:
            for q in range(_NQ):
                qsl = pl.ds(q * _QW, _QW)
                pltpu.make_async_copy(acc_ref.at[:, qsl], out_ref.at[:, qsl],
                                      sem.at[q]).wait()

    @pl.when(e == 0)
    def _first():
        x32 = x_ref[...]
        logits = jnp.dot(x32, wa_ref[...],
                         preferred_element_type=jnp.float32) + bias_ref[...]
        w = jax.nn.softmax(logits, axis=-1)
        w_ref[...] = w
        xb = x32.astype(jnp.bfloat16)
        xb_ref[...] = xb
        _steps(True, False, w=w, xb=xb)

    @pl.when(jnp.logical_and(e > 0, e < _E - 1))
    def _main():
        _steps(False, False)

    @pl.when(e == _E - 1)
    def _last():
        _steps(False, True)


def kernel(x, W_attn, b_attn, adaptive_bias, W_s, b_s):
    bias = (b_attn + adaptive_bias).reshape(1, _E)
    return pl.pallas_call(
        _fused_kernel,
        grid=(_E,),
        in_specs=[
            pl.BlockSpec((_T, _D), lambda e: (0, 0)),
            pl.BlockSpec((_D, _E), lambda e: (0, 0)),
            pl.BlockSpec((1, _E), lambda e: (0, 0)),
            pl.BlockSpec((1, _D, _D), lambda e: (e, 0, 0)),
            pl.BlockSpec((1, 1, _D), lambda e: (e, 0, 0)),
        ],
        out_specs=pl.BlockSpec(memory_space=pltpu.MemorySpace.HBM),
        out_shape=jax.ShapeDtypeStruct((_T, _D), jnp.float32),
        scratch_shapes=[
            pltpu.VMEM((_T, _D), jnp.bfloat16),
            pltpu.VMEM((_T, _E), jnp.float32),
            pltpu.VMEM((_T, _D), jnp.float32),
            pltpu.SemaphoreType.DMA((_NQ,)),
        ],
        compiler_params=pltpu.CompilerParams(
            dimension_semantics=("arbitrary",),
        ),
    )(x, W_attn, bias, W_s, b_s.reshape(_E, 1, _D))
